# baseline (speedup 1.0000x reference)
import functools

import jax
import jax.numpy as jnp
from jax import lax
from jax.experimental import pallas as pl
from jax.experimental.pallas import tpu as pltpu
from jax.experimental.pallas import tpu_sc as plsc

_L = 16


@functools.cache
def _build(B, D, R, NBUF):
    info = plsc.get_sparse_core_info()
    NC, NS = info.num_cores, info.num_subcores
    NW = NC * NS
    assert B % (NW * R) == 0 and D % _L == 0
    RW = B // NW
    NCHUNK = RW // R
    assert NCHUNK % NBUF == 0

    mesh = plsc.VectorSubcoreMesh(core_axis_name="c", subcore_axis_name="s")

    @functools.partial(
        pl.kernel,
        out_type=jax.ShapeDtypeStruct((B, D), jnp.float32),
        mesh=mesh,
        compiler_params=pltpu.CompilerParams(needs_layout_passes=False,
                                             disable_bounds_checks=True),
        scratch_types=[
            pltpu.VMEM((D,), jnp.int32),
            pltpu.VMEM((D,), jnp.int32),
            [pltpu.VMEM((R, D), jnp.float32)] * NBUF,
            [pltpu.VMEM((R, D), jnp.float32)] * NBUF,
            [pltpu.SemaphoreType.DMA] * NBUF,
            [pltpu.SemaphoreType.DMA] * NBUF,
        ],
    )
    def permute_kernel(x_hbm, gsrc_hbm, sig_hbm, out_hbm, gsrc_v, sig_v,
                       in_v, out_v, in_sems, out_sems):
        wid = lax.axis_index("s") * NC + lax.axis_index("c")
        base = wid * RW
        pltpu.sync_copy(gsrc_hbm, gsrc_v)
        pltpu.sync_copy(sig_hbm, sig_v)

        def start_in(b, c):
            pltpu.async_copy(x_hbm.at[pl.ds(base + c * R, R)], in_v[b],
                             in_sems[b])

        def start_out(b, c):
            pltpu.async_copy(out_v[b], out_hbm.at[pl.ds(base + c * R, R)],
                             out_sems[b])

        def wait_in(b):
            pltpu.make_async_copy(x_hbm.at[pl.ds(base, R)], in_v[b],
                                  in_sems[b]).wait()

        def wait_out(b):
            pltpu.make_async_copy(out_v[b], out_hbm.at[pl.ds(base, R)],
                                  out_sems[b]).wait()

        for b in range(NBUF):
            start_in(b, b)

        def outer(c2, carry):
            for b in range(NBUF):
                c = NBUF * c2 + b
                wait_in(b)

                @pl.when(c2 >= 1)
                def _():
                    wait_out(b)

                def col_body(k, carry2):
                    gv = gsrc_v[pl.ds(k * _L, _L)]
                    sv = sig_v[pl.ds(k * _L, _L)]
                    for r in range(R):
                        rvec = jnp.full((_L,), r, jnp.int32)
                        v = plsc.load_gather(in_v[b], [rvec, gv])
                        plsc.store_scatter(out_v[b], [rvec, sv], v)
                    return carry2

                lax.fori_loop(0, D // _L, col_body, 0)
                start_out(b, c)

                @pl.when(c2 < NCHUNK // NBUF - 1)
                def _():
                    start_in(b, c + NBUF)
            return carry

        lax.fori_loop(0, NCHUNK // NBUF, outer, 0)
        for b in range(NBUF):
            wait_out(b)

    return permute_kernel


def kernel(x, perm):
    B, D = x.shape
    perm32 = perm.astype(jnp.int32)
    nb = _L
    cls = D // nb
    j = jnp.arange(D, dtype=jnp.int32)
    key = (perm32 % nb) * (D * nb) + (j % nb) * D + j
    order2 = jnp.argsort(key).astype(jnp.int32).reshape(nb, cls)
    shift = (jnp.arange(cls, dtype=jnp.int32)[None, :]
             + (cls // nb) * jnp.arange(nb, dtype=jnp.int32)[:, None]) % cls
    grouped = jnp.take_along_axis(order2, shift, axis=1)
    sigma = grouped.T.reshape(D)
    gsrc = perm32[sigma]
    z = _build(B, D, 8, 2)(x, gsrc, sigma)
    return (z, 0)


# speedup vs baseline: 1.2575x; 1.2575x over previous
import functools

import jax
import jax.numpy as jnp
from jax import lax
from jax.experimental import pallas as pl
from jax.experimental.pallas import tpu as pltpu
from jax.experimental.pallas import tpu_sc as plsc

_L = 16


@functools.cache
def _build(B, D, R, NBUF):
    info = plsc.get_sparse_core_info()
    NC, NS = info.num_cores, info.num_subcores
    NW = NC * NS
    assert B % (NW * R) == 0 and D % _L == 0
    RW = B // NW
    NCHUNK = RW // R
    assert NCHUNK % NBUF == 0

    mesh = plsc.VectorSubcoreMesh(core_axis_name="c", subcore_axis_name="s")

    @functools.partial(
        pl.kernel,
        out_type=jax.ShapeDtypeStruct((B, D), jnp.float32),
        mesh=mesh,
        compiler_params=pltpu.CompilerParams(needs_layout_passes=False,
                                             disable_bounds_checks=True),
        scratch_types=[
            pltpu.VMEM((D,), jnp.int32),
            [pltpu.VMEM((R, D), jnp.float32)] * NBUF,
            [pltpu.VMEM((R, D), jnp.float32)] * NBUF,
            [pltpu.SemaphoreType.DMA] * NBUF,
            [pltpu.SemaphoreType.DMA] * NBUF,
        ],
    )
    def permute_kernel(x_hbm, iperm_hbm, out_hbm, iperm_v,
                       in_v, out_v, in_sems, out_sems):
        wid = lax.axis_index("s") * NC + lax.axis_index("c")
        base = wid * RW
        pltpu.sync_copy(iperm_hbm, iperm_v)

        def start_in(b, c):
            pltpu.async_copy(x_hbm.at[pl.ds(base + c * R, R)], in_v[b],
                             in_sems[b])

        def start_out(b, c):
            pltpu.async_copy(out_v[b], out_hbm.at[pl.ds(base + c * R, R)],
                             out_sems[b])

        def wait_in(b):
            pltpu.make_async_copy(x_hbm.at[pl.ds(base, R)], in_v[b],
                                  in_sems[b]).wait()

        def wait_out(b):
            pltpu.make_async_copy(out_v[b], out_hbm.at[pl.ds(base, R)],
                                  out_sems[b]).wait()

        for b in range(NBUF):
            start_in(b, b)

        def outer(c2, carry):
            for b in range(NBUF):
                c = NBUF * c2 + b
                wait_in(b)

                @pl.when(c2 >= 1)
                def _():
                    wait_out(b)

                def col_body(k, carry2):
                    sv = iperm_v[pl.ds(k * _L, _L)]
                    for r in range(R):
                        rvec = jnp.full((_L,), r, jnp.int32)
                        v = in_v[b][r, pl.ds(k * _L, _L)]
                        plsc.store_scatter(out_v[b], [rvec, sv], v)
                    return carry2

                lax.fori_loop(0, D // _L, col_body, 0)
                start_out(b, c)

                @pl.when(c2 < NCHUNK // NBUF - 1)
                def _():
                    start_in(b, c + NBUF)
            return carry

        lax.fori_loop(0, NCHUNK // NBUF, outer, 0)
        for b in range(NBUF):
            wait_out(b)

    return permute_kernel


def kernel(x, perm):
    B, D = x.shape
    iperm = jnp.argsort(perm).astype(jnp.int32)
    z = _build(B, D, 8, 2)(x, iperm)
    return (z, 0)
